# baseline (speedup 1.0000x reference)
import functools

import jax
import jax.numpy as jnp
from jax import lax
from jax.experimental import pallas as pl
from jax.experimental.pallas import tpu as pltpu
from jax.experimental.pallas import tpu_sc as plsc

B, N, E = 2, 50000, 800000
NC, NS, L = 2, 16, 16
SB = 128
CR = 16
CJ = CR * 128 // (2 * SB)
EROW = E * 2 // 128
NCH = E * 2 // (CR * 128)
TAILE = E - NCH * CR * 64
NG = -(-NCH // NS)
SG = N // 10
FN = 3200
FLAST = N - 15 * FN
DW = 8000
DWK = (B * N * 2) // DW
EPS = 1e-8
DIV_W, LAP_W, DATA_W = 1.0, 0.1, 1.0


def _rsqrt(z):
    ii = lax.bitcast_convert_type(z, jnp.int32)
    ii = jnp.int32(0x5F3759DF) - (ii >> 1)
    y = lax.bitcast_convert_type(ii, jnp.float32)
    for _ in range(3):
        y = y * (1.5 - 0.5 * z * y * y)
    return y


def _sc_body_real(xf, uf, gf, exf, tlf, zf, part_hbm,
                  t0, t1, t2, t3, a0, a1, a2, a3, a4,
                  idx_i, idx_j, ebuf, tb,
                  g0, g1, g2, g3, g4, g5, g6, g7,
                  g8, g9, g10, g11, g12, g13, g14, g15,
                  b0, b1, b2, b3, b4, b5, b6, b7,
                  b8, b9, b10, b11, b12, b13, b14, b15,
                  stg, d0, d1, f0, f1, f2, f3, f4, ub, gbv, pb,
                  sg0, sg1, ss0, ss1):
    c = lax.axis_index("c")
    s = lax.axis_index("s")
    wid = c * NS + s
    iot = lax.iota(jnp.int32, L)
    tabs = (t0, t1, t2, t3)
    accs = (a0, a1, a2, a3, a4)
    gb = (g0, g1, g2, g3, g4, g5, g6, g7,
          g8, g9, g10, g11, g12, g13, g14, g15)
    ubs = (b0, b1, b2, b3, b4, b5, b6, b7,
           b8, b9, b10, b11, b12, b13, b14, b15)
    fins = (f0, f1, f2, f3, f4)

    def _deint(n):
        def body(k, carry):
            pos = 2 * (k * L) + 2 * iot
            d0[pl.ds(k * L, L)] = plsc.load_gather(stg, [pos])
            d1[pl.ds(k * L, L)] = plsc.load_gather(stg, [pos + 1])
            return carry

        lax.fori_loop(0, n // L, body, 0)
        tail = n - (n // L) * L
        if tail:
            rows0 = (n // L) * L + iot
            valid = rows0 < n
            rows = jnp.minimum(rows0, n - 1)
            v0 = plsc.load_gather(stg, [2 * rows])
            v1 = plsc.load_gather(stg, [2 * rows + 1])
            plsc.store_scatter(d0, [rows], v0, mask=valid)
            plsc.store_scatter(d1, [rows], v1, mask=valid)

    @pl.when(s < 10)
    def _stage():
        pltpu.sync_copy(xf.at[pl.ds(2 * (c * N + s * SG), 2 * SG)], stg)
        _deint(SG)
        pltpu.sync_copy(d0, t0.at[pl.ds(s * SG, SG)])
        pltpu.sync_copy(d1, t1.at[pl.ds(s * SG, SG)])
        pltpu.sync_copy(uf.at[pl.ds(2 * (c * N + s * SG), 2 * SG)], stg)
        _deint(SG)
        pltpu.sync_copy(d0, t2.at[pl.ds(s * SG, SG)])
        pltpu.sync_copy(d1, t3.at[pl.ds(s * SG, SG)])
        pltpu.sync_copy(zf, d0)
        for t in range(5):
            pltpu.sync_copy(d0, accs[t].at[pl.ds(s * SG, SG)])

    plsc.subcore_barrier()

    sgs = (sg0, sg1)
    sss = (ss0, ss1)
    dummy = xf.at[pl.ds(0, SB)]

    def issue_gathers(p, j):
        ir = idx_i.at[j]
        jr = idx_j.at[j]
        for t in range(4):
            pltpu.async_copy(tabs[t].at[ir], gb[8 * p + t], sgs[p])
            pltpu.async_copy(tabs[t].at[jr], gb[8 * p + 4 + t], sgs[p])

    def wait_gathers(p):
        for t in range(8):
            pltpu.make_async_copy(dummy, gb[8 * p + t], sgs[p]).wait()

    def issue_scatters(p, j):
        ir = idx_i.at[j]
        jr = idx_j.at[j]
        o = 8 * p
        for t in range(3):
            pltpu.async_copy(ubs[o + t], accs[t].at[ir], sss[p], add=True)
            pltpu.async_copy(ubs[o + 5 + t], accs[t].at[jr], sss[p], add=True)
        for t in range(3, 5):
            pltpu.async_copy(ubs[o + t], accs[t].at[ir], sss[p], add=True)
            pltpu.async_copy(ubs[o + t], accs[t].at[jr], sss[p], add=True)

    def wait_scatters(p):
        for t in range(10):
            pltpu.make_async_copy(dummy, ubs[8 * p], sss[p]).wait()

    def compute(p):
        o = 8 * p
        for k in range(SB // L):
            sl = pl.ds(k * L, L)
            x0i = gb[o + 0][sl]; x1i = gb[o + 1][sl]
            u0i = gb[o + 2][sl]; u1i = gb[o + 3][sl]
            x0j = gb[o + 4][sl]; x1j = gb[o + 5][sl]
            u0j = gb[o + 6][sl]; u1j = gb[o + 7][sl]
            dx = x0j - x0i
            dy = x1j - x1i
            len2 = dx * dx + dy * dy + EPS
            r = _rsqrt(len2)
            wg = r * r
            rl = _rsqrt(len2 + EPS)
            wl = rl * rl
            du0 = u0j - u0i
            du1 = u1j - u1i
            divc = wg * r * (du0 * dx + du1 * dy)
            lap0 = wl * du0
            lap1 = wl * du1
            ubs[o + 0][sl] = divc
            ubs[o + 1][sl] = lap0
            ubs[o + 2][sl] = lap1
            ubs[o + 3][sl] = wg
            ubs[o + 4][sl] = wl
            ubs[o + 5][sl] = -divc
            ubs[o + 6][sl] = -lap0
            ubs[o + 7][sl] = -lap1

    def _pipe_pair(jj, carry2):
        j = 2 * jj
        wait_gathers(0)
        issue_gathers(1, j + 1)

        @pl.when(jj > 0)
        def _():
            wait_scatters(0)

        compute(0)
        issue_scatters(0, j)

        wait_gathers(1)

        @pl.when(jj < CJ // 2 - 1)
        def _():
            issue_gathers(0, j + 2)

        @pl.when(jj > 0)
        def _():
            wait_scatters(1)

        compute(1)
        issue_scatters(1, j + 1)
        return carry2

    def _deint_idx(j, carry):
        zv = jnp.zeros((L,), jnp.int32)
        jrow = zv + j
        for k in range(SB // L):
            row = zv + (2 * j + k // 4)
            col = 32 * (k % 4) + 2 * iot
            iv = plsc.load_gather(ebuf, [row, col])
            jv = plsc.load_gather(ebuf, [row, col + 1])
            cols = k * L + iot
            plsc.store_scatter(idx_i, [jrow, cols], iv)
            plsc.store_scatter(idx_j, [jrow, cols], jv)
        return carry

    def edge_chunk(g, carry):
        h = g * NS + s

        @pl.when(h < NCH)
        def _chunk():
            pltpu.sync_copy(exf.at[c, pl.ds(h * CR, CR)], ebuf)
            lax.fori_loop(0, CJ, _deint_idx, 0)
            issue_gathers(0, 0)
            lax.fori_loop(0, CJ // 2, _pipe_pair, 0)
            wait_scatters(0)
            wait_scatters(1)

        return carry

    lax.fori_loop(0, NG, edge_chunk, 0)

    @pl.when(s == 15)
    def _tail():
        pltpu.sync_copy(tlf.at[pl.ds(c * 2 * TAILE, 2 * TAILE)], tb)
        for j2 in range(TAILE // SB):
            j2row = jnp.full((L,), j2, jnp.int32)
            for k in range(SB // L):
                pos = 2 * (j2 * SB + k * L) + 2 * iot
                iv = plsc.load_gather(tb, [pos])
                jv = plsc.load_gather(tb, [pos + 1])
                cols = k * L + iot
                plsc.store_scatter(idx_i, [j2row, cols], iv)
                plsc.store_scatter(idx_j, [j2row, cols], jv)
        for j2 in range(TAILE // SB):
            issue_gathers(0, j2)
            wait_gathers(0)
            compute(0)
            issue_scatters(0, j2)
            wait_scatters(0)

    plsc.subcore_barrier()

    @pl.when(s < 15)
    def _rb_full():
        for t in range(5):
            pltpu.sync_copy(accs[t].at[pl.ds(s * FN, FN)], fins[t])

    @pl.when(s == 15)
    def _rb_last():
        for t in range(5):
            pltpu.sync_copy(accs[t].at[pl.ds(15 * FN, FLAST)],
                            fins[t].at[pl.ds(0, FLAST)])

    limit = jnp.where(s < 15, FN, FLAST)
    zero = jnp.zeros((L,), jnp.float32)

    def nodef(t, carry):
        sdv, slp = carry
        rows0 = t * L + iot
        valid = (rows0 < limit).astype(jnp.float32)
        sl = pl.ds(t * L, L)
        a0v = f0[sl]; a1v = f1[sl]; a2v = f2[sl]
        a3v = f3[sl]; a4v = f4[sl]
        dv = a0v / (a3v + EPS)
        l0 = a1v / (a4v + EPS)
        l1 = a2v / (a4v + EPS)
        return (sdv + valid * dv * dv,
                slp + valid * (l0 * l0 + l1 * l1))

    sdv, slp = lax.fori_loop(0, FN // L, nodef, (zero, zero))

    base = jnp.minimum(wid, DWK - 1) * DW
    pltpu.sync_copy(uf.at[pl.ds(base, DW)], ub)
    pltpu.sync_copy(gf.at[pl.ds(base, DW)], gbv)

    def dataf(t, acc):
        dd = ub[pl.ds(t * L, L)] - gbv[pl.ds(t * L, L)]
        return acc + dd * dd

    sdat = lax.fori_loop(0, DW // L, dataf, zero)
    live = jnp.where(wid < DWK, 1.0, 0.0).astype(jnp.float32)

    pb[0] = sdv * (DIV_W / (B * N))
    pb[1] = slp * (LAP_W / (B * N * 2))
    pb[2] = sdat * live * (DATA_W / (B * N * 2))
    pltpu.sync_copy(pb, part_hbm.at[wid])


@functools.cache
def _build_sc_kernel():
    return pl.kernel(
        _sc_body_real,
        out_type=jax.ShapeDtypeStruct((NC * NS, 3, L), jnp.float32),
        mesh=plsc.VectorSubcoreMesh(
            core_axis_name="c", subcore_axis_name="s",
            num_cores=NC, num_subcores=NS),
        compiler_params=pltpu.CompilerParams(needs_layout_passes=False),
        scratch_types=(
            [pltpu.VMEM_SHARED((N,), jnp.float32)] * 4
            + [pltpu.VMEM_SHARED((N,), jnp.float32)] * 5
            + [pltpu.VMEM((CJ, SB), jnp.int32)] * 2
            + [pltpu.VMEM((CR, 128), jnp.int32)]
            + [pltpu.VMEM((2 * TAILE,), jnp.int32)]
            + [pltpu.VMEM((SB,), jnp.float32)] * 16
            + [pltpu.VMEM((SB,), jnp.float32)] * 16
            + [pltpu.VMEM((2 * SG,), jnp.float32)]
            + [pltpu.VMEM((SG,), jnp.float32)] * 2
            + [pltpu.VMEM((FN,), jnp.float32)] * 5
            + [pltpu.VMEM((DW,), jnp.float32)] * 2
            + [pltpu.VMEM((3, L), jnp.float32)]
            + [pltpu.SemaphoreType.DMA] * 4
        ),
    )


def _sum_body(x_ref, o_ref):
    o_ref[0, 0] = jnp.sum(x_ref[...])


def _final_sum(x):
    return pl.pallas_call(
        _sum_body,
        out_shape=jax.ShapeDtypeStruct((1, 1), jnp.float32),
        out_specs=pl.BlockSpec(memory_space=pltpu.SMEM),
    )(x)


@jax.jit
def kernel(x, u, u_gt, edges):
    xf = x.reshape(-1)
    uf = u.reshape(-1)
    gf = u_gt.reshape(-1)
    exf = edges.reshape(B, EROW, 128)
    tlf = edges[:, E - TAILE:, :].reshape(-1)
    zf = jnp.zeros((SG,), jnp.float32)
    part = _build_sc_kernel()(xf, uf, gf, exf, tlf, zf)
    return _final_sum(part.reshape(12, 128))[0, 0]


# speedup vs baseline: 3.8257x; 3.8257x over previous
import functools

import jax
import jax.numpy as jnp
from jax import lax
from jax.experimental import pallas as pl
from jax.experimental.pallas import tpu as pltpu
from jax.experimental.pallas import tpu_sc as plsc

B, N, E = 2, 50000, 800000
NC, NS, L = 2, 16, 16
SB = 128
CJ = 8
EIR = E // SB
EIRP = EIR + 6
NP = N + 8
NCH = EIRP // CJ
NG = -(-NCH // NS)
SG = N // 10
FN = 3200
FLAST = N - 15 * FN
DW = 8000
DWK = (B * N * 2) // DW
EPS = 1e-8
DIV_W, LAP_W, DATA_W = 1.0, 0.1, 1.0


def _rsqrt(z):
    ii = lax.bitcast_convert_type(z, jnp.int32)
    ii = jnp.int32(0x5F3759DF) - (ii >> 1)
    y = lax.bitcast_convert_type(ii, jnp.float32)
    for _ in range(3):
        y = y * (1.5 - 0.5 * z * y * y)
    return y


def _sc_body_real(xf, uf, gf, eip, ejp, zf, part_hbm,
                  t0, t1, t2, t3, a0, a1, a2, a3, a4,
                  idx_i, idx_j,
                  g0, g1, g2, g3, g4, g5, g6, g7,
                  g8, g9, g10, g11, g12, g13, g14, g15,
                  b0, b1, b2, b3, b4, b5, b6, b7,
                  b8, b9, b10, b11, b12, b13, b14, b15,
                  stg, d0, d1, f0, f1, f2, f3, f4, ub, gbv, pb,
                  sg0, sg1, ss0, ss1):
    c = lax.axis_index("c")
    s = lax.axis_index("s")
    wid = c * NS + s
    iot = lax.iota(jnp.int32, L)
    tabs = (t0, t1, t2, t3)
    accs = (a0, a1, a2, a3, a4)
    gb = (g0, g1, g2, g3, g4, g5, g6, g7,
          g8, g9, g10, g11, g12, g13, g14, g15)
    ubs = (b0, b1, b2, b3, b4, b5, b6, b7,
           b8, b9, b10, b11, b12, b13, b14, b15)
    fins = (f0, f1, f2, f3, f4)

    def _deint(n):
        def body(k, carry):
            pos = 2 * (k * L) + 2 * iot
            d0[pl.ds(k * L, L)] = plsc.load_gather(stg, [pos])
            d1[pl.ds(k * L, L)] = plsc.load_gather(stg, [pos + 1])
            return carry

        lax.fori_loop(0, n // L, body, 0)
        tail = n - (n // L) * L
        if tail:
            rows0 = (n // L) * L + iot
            valid = rows0 < n
            rows = jnp.minimum(rows0, n - 1)
            v0 = plsc.load_gather(stg, [2 * rows])
            v1 = plsc.load_gather(stg, [2 * rows + 1])
            plsc.store_scatter(d0, [rows], v0, mask=valid)
            plsc.store_scatter(d1, [rows], v1, mask=valid)

    @pl.when(s < 10)
    def _stage():
        pltpu.sync_copy(xf.at[pl.ds(2 * (c * N + s * SG), 2 * SG)], stg)
        _deint(SG)
        pltpu.sync_copy(d0, t0.at[pl.ds(s * SG, SG)])
        pltpu.sync_copy(d1, t1.at[pl.ds(s * SG, SG)])
        pltpu.sync_copy(uf.at[pl.ds(2 * (c * N + s * SG), 2 * SG)], stg)
        _deint(SG)
        pltpu.sync_copy(d0, t2.at[pl.ds(s * SG, SG)])
        pltpu.sync_copy(d1, t3.at[pl.ds(s * SG, SG)])
        pltpu.sync_copy(zf, d0)
        for t in range(5):
            pltpu.sync_copy(d0, accs[t].at[pl.ds(s * SG, SG)])

    @pl.when(s == 10)
    def _stage_pad():
        pltpu.sync_copy(zf, d0)
        for t in range(4):
            pltpu.sync_copy(d0.at[pl.ds(0, 8)], tabs[t].at[pl.ds(N, 8)])
        for t in range(5):
            pltpu.sync_copy(d0.at[pl.ds(0, 8)], accs[t].at[pl.ds(N, 8)])

    plsc.subcore_barrier()

    sgs = (sg0, sg1)
    sss = (ss0, ss1)
    dummy = xf.at[pl.ds(0, SB)]

    def issue_gathers(p, j):
        ir = idx_i.at[j]
        jr = idx_j.at[j]
        for t in range(4):
            pltpu.async_copy(tabs[t].at[ir], gb[8 * p + t], sgs[p])
            pltpu.async_copy(tabs[t].at[jr], gb[8 * p + 4 + t], sgs[p])

    def wait_gathers(p):
        for t in range(8):
            pltpu.make_async_copy(dummy, gb[8 * p + t], sgs[p]).wait()

    def issue_scatters(p, j):
        ir = idx_i.at[j]
        jr = idx_j.at[j]
        o = 8 * p
        for t in range(3):
            pltpu.async_copy(ubs[o + t], accs[t].at[ir], sss[p], add=True)
            pltpu.async_copy(ubs[o + 5 + t], accs[t].at[jr], sss[p], add=True)
        for t in range(3, 5):
            pltpu.async_copy(ubs[o + t], accs[t].at[ir], sss[p], add=True)
            pltpu.async_copy(ubs[o + t], accs[t].at[jr], sss[p], add=True)

    def wait_scatters(p):
        for t in range(10):
            pltpu.make_async_copy(dummy, ubs[8 * p], sss[p]).wait()

    def compute(p):
        o = 8 * p
        for k in range(SB // L):
            sl = pl.ds(k * L, L)
            x0i = gb[o + 0][sl]; x1i = gb[o + 1][sl]
            u0i = gb[o + 2][sl]; u1i = gb[o + 3][sl]
            x0j = gb[o + 4][sl]; x1j = gb[o + 5][sl]
            u0j = gb[o + 6][sl]; u1j = gb[o + 7][sl]
            dx = x0j - x0i
            dy = x1j - x1i
            len2 = dx * dx + dy * dy + EPS
            r = _rsqrt(len2)
            wg = r * r
            rl = _rsqrt(len2 + EPS)
            wl = rl * rl
            du0 = u0j - u0i
            du1 = u1j - u1i
            divc = wg * r * (du0 * dx + du1 * dy)
            lap0 = wl * du0
            lap1 = wl * du1
            ubs[o + 0][sl] = divc
            ubs[o + 1][sl] = lap0
            ubs[o + 2][sl] = lap1
            ubs[o + 3][sl] = wg
            ubs[o + 4][sl] = wl
            ubs[o + 5][sl] = -divc
            ubs[o + 6][sl] = -lap0
            ubs[o + 7][sl] = -lap1

    def _pipe_pair(jj, carry2):
        j = 2 * jj
        wait_gathers(0)
        issue_gathers(1, j + 1)

        @pl.when(jj > 0)
        def _():
            wait_scatters(0)

        compute(0)
        issue_scatters(0, j)

        wait_gathers(1)

        @pl.when(jj < CJ // 2 - 1)
        def _():
            issue_gathers(0, j + 2)

        @pl.when(jj > 0)
        def _():
            wait_scatters(1)

        compute(1)
        issue_scatters(1, j + 1)
        return carry2

    def edge_chunk(g, carry):
        h = g * NS + s

        @pl.when(h < NCH)
        def _chunk():
            pltpu.sync_copy(eip.at[c, pl.ds(h * CJ, CJ)], idx_i)
            pltpu.sync_copy(ejp.at[c, pl.ds(h * CJ, CJ)], idx_j)
            issue_gathers(0, 0)
            lax.fori_loop(0, CJ // 2, _pipe_pair, 0)
            wait_scatters(0)
            wait_scatters(1)

        return carry

    lax.fori_loop(0, NG, edge_chunk, 0)
    plsc.subcore_barrier()

    @pl.when(s < 15)
    def _rb_full():
        for t in range(5):
            pltpu.sync_copy(accs[t].at[pl.ds(s * FN, FN)], fins[t])

    @pl.when(s == 15)
    def _rb_last():
        for t in range(5):
            pltpu.sync_copy(accs[t].at[pl.ds(15 * FN, FLAST)],
                            fins[t].at[pl.ds(0, FLAST)])

    limit = jnp.where(s < 15, FN, FLAST)
    zero = jnp.zeros((L,), jnp.float32)

    def nodef(t, carry):
        sdv, slp = carry
        rows0 = t * L + iot
        valid = (rows0 < limit).astype(jnp.float32)
        sl = pl.ds(t * L, L)
        a0v = f0[sl]; a1v = f1[sl]; a2v = f2[sl]
        a3v = f3[sl]; a4v = f4[sl]
        dv = a0v / (a3v + EPS)
        l0 = a1v / (a4v + EPS)
        l1 = a2v / (a4v + EPS)
        return (sdv + valid * dv * dv,
                slp + valid * (l0 * l0 + l1 * l1))

    sdv, slp = lax.fori_loop(0, FN // L, nodef, (zero, zero))

    base = jnp.minimum(wid, DWK - 1) * DW
    pltpu.sync_copy(uf.at[pl.ds(base, DW)], ub)
    pltpu.sync_copy(gf.at[pl.ds(base, DW)], gbv)

    def dataf(t, acc):
        dd = ub[pl.ds(t * L, L)] - gbv[pl.ds(t * L, L)]
        return acc + dd * dd

    sdat = lax.fori_loop(0, DW // L, dataf, zero)
    live = jnp.where(wid < DWK, 1.0, 0.0).astype(jnp.float32)

    pb[0] = sdv * (DIV_W / (B * N))
    pb[1] = slp * (LAP_W / (B * N * 2))
    pb[2] = sdat * live * (DATA_W / (B * N * 2))
    pltpu.sync_copy(pb, part_hbm.at[wid])


@functools.cache
def _build_sc_kernel():
    return pl.kernel(
        _sc_body_real,
        out_type=jax.ShapeDtypeStruct((NC * NS, 3, L), jnp.float32),
        mesh=plsc.VectorSubcoreMesh(
            core_axis_name="c", subcore_axis_name="s",
            num_cores=NC, num_subcores=NS),
        compiler_params=pltpu.CompilerParams(needs_layout_passes=False),
        scratch_types=(
            [pltpu.VMEM_SHARED((NP,), jnp.float32)] * 4
            + [pltpu.VMEM_SHARED((NP,), jnp.float32)] * 5
            + [pltpu.VMEM((CJ, SB), jnp.int32)] * 2
            + [pltpu.VMEM((SB,), jnp.float32)] * 16
            + [pltpu.VMEM((SB,), jnp.float32)] * 16
            + [pltpu.VMEM((2 * SG,), jnp.float32)]
            + [pltpu.VMEM((SG,), jnp.float32)] * 2
            + [pltpu.VMEM((FN,), jnp.float32)] * 5
            + [pltpu.VMEM((DW,), jnp.float32)] * 2
            + [pltpu.VMEM((3, L), jnp.float32)]
            + [pltpu.SemaphoreType.DMA] * 4
        ),
    )


def _sum_body(x_ref, o_ref):
    o_ref[0, 0] = jnp.sum(x_ref[...])


def _final_sum(x):
    return pl.pallas_call(
        _sum_body,
        out_shape=jax.ShapeDtypeStruct((1, 1), jnp.float32),
        out_specs=pl.BlockSpec(memory_space=pltpu.SMEM),
    )(x)


@jax.jit
def kernel(x, u, u_gt, edges):
    xf = x.reshape(-1)
    uf = u.reshape(-1)
    gf = u_gt.reshape(-1)
    eip = jnp.pad(edges[..., 0].reshape(B, EIR, SB),
                  ((0, 0), (0, EIRP - EIR), (0, 0)), constant_values=N)
    ejp = jnp.pad(edges[..., 1].reshape(B, EIR, SB),
                  ((0, 0), (0, EIRP - EIR), (0, 0)), constant_values=N)
    zf = jnp.zeros((SG,), jnp.float32)
    part = _build_sc_kernel()(xf, uf, gf, eip, ejp, zf)
    return _final_sum(part.reshape(12, 128))[0, 0]


# speedup vs baseline: 4.3266x; 1.1309x over previous
import functools

import jax
import jax.numpy as jnp
from jax import lax
from jax.experimental import pallas as pl
from jax.experimental.pallas import tpu as pltpu
from jax.experimental.pallas import tpu_sc as plsc

B, N, E = 2, 50000, 800000
NC, NS, L = 2, 16, 16
SB = 128
CJ = 16
EIR = E // SB
EIRP = EIR + 6
NP = N + 8
NCH = EIRP // CJ
NG = -(-NCH // NS)
SG = N // 10
FN = 3200
FLAST = N - 15 * FN
DW = 8000
DWK = (B * N * 2) // DW
EPS = 1e-8
DIV_W, LAP_W, DATA_W = 1.0, 0.1, 1.0


X0, U0, G0 = 0, 2 * B * N, 4 * B * N


def _rsqrt(z):
    ii = lax.bitcast_convert_type(z, jnp.int32)
    ii = jnp.int32(0x5F3759DF) - (ii >> 1)
    y = lax.bitcast_convert_type(ii, jnp.float32)
    for _ in range(2):
        y = y * (1.5 - 0.5 * z * y * y)
    return y


def _sc_body_real(xug, eip, ejp, zf, part_hbm,
                  t0, t1, t2, t3, a0, a1, a2, a3,
                  idx_i, idx_j,
                  g0, g1, g2, g3, g4, g5, g6, g7,
                  g8, g9, g10, g11, g12, g13, g14, g15,
                  b0, b1, b2, b3, b4, b5, b6,
                  b7, b8, b9, b10, b11, b12, b13,
                  stg, d0, d1, f0, f1, f2, f3, ub, gbv, pb,
                  sg0, sg1, ss0, ss1):
    c = lax.axis_index("c")
    s = lax.axis_index("s")
    wid = c * NS + s
    iot = lax.iota(jnp.int32, L)
    tabs = (t0, t1, t2, t3)
    accs = (a0, a1, a2, a3)
    gb = (g0, g1, g2, g3, g4, g5, g6, g7,
          g8, g9, g10, g11, g12, g13, g14, g15)
    ubs = (b0, b1, b2, b3, b4, b5, b6,
           b7, b8, b9, b10, b11, b12, b13)
    fins = (f0, f1, f2, f3)

    def _deint(n):
        def body(k, carry):
            pos = 2 * (k * L) + 2 * iot
            d0[pl.ds(k * L, L)] = plsc.load_gather(stg, [pos])
            d1[pl.ds(k * L, L)] = plsc.load_gather(stg, [pos + 1])
            return carry

        lax.fori_loop(0, n // L, body, 0)
        tail = n - (n // L) * L
        if tail:
            rows0 = (n // L) * L + iot
            valid = rows0 < n
            rows = jnp.minimum(rows0, n - 1)
            v0 = plsc.load_gather(stg, [2 * rows])
            v1 = plsc.load_gather(stg, [2 * rows + 1])
            plsc.store_scatter(d0, [rows], v0, mask=valid)
            plsc.store_scatter(d1, [rows], v1, mask=valid)

    @pl.when(s < 10)
    def _stage():
        pltpu.sync_copy(xug.at[pl.ds(X0 + 2 * (c * N + s * SG), 2 * SG)], stg)
        _deint(SG)
        pltpu.sync_copy(d0, t0.at[pl.ds(s * SG, SG)])
        pltpu.sync_copy(d1, t1.at[pl.ds(s * SG, SG)])
        pltpu.sync_copy(xug.at[pl.ds(U0 + 2 * (c * N + s * SG), 2 * SG)], stg)
        _deint(SG)
        pltpu.sync_copy(d0, t2.at[pl.ds(s * SG, SG)])
        pltpu.sync_copy(d1, t3.at[pl.ds(s * SG, SG)])
        pltpu.sync_copy(zf, d0)
        for t in range(4):
            pltpu.sync_copy(d0, accs[t].at[pl.ds(s * SG, SG)])

    @pl.when(s == 10)
    def _stage_pad():
        pltpu.sync_copy(zf, d0)
        for t in range(4):
            pltpu.sync_copy(d0.at[pl.ds(0, 8)], tabs[t].at[pl.ds(N, 8)])
        for t in range(4):
            pltpu.sync_copy(d0.at[pl.ds(0, 8)], accs[t].at[pl.ds(N, 8)])

    plsc.subcore_barrier()

    sgs = (sg0, sg1)
    sss = (ss0, ss1)
    dummy = xug.at[pl.ds(0, SB)]

    def issue_gathers(p, j):
        ir = idx_i.at[j]
        jr = idx_j.at[j]
        for t in range(4):
            pltpu.async_copy(tabs[t].at[ir], gb[8 * p + t], sgs[p])
            pltpu.async_copy(tabs[t].at[jr], gb[8 * p + 4 + t], sgs[p])

    def wait_gathers(p):
        for t in range(8):
            pltpu.make_async_copy(dummy, gb[8 * p + t], sgs[p]).wait()

    def issue_scatters(p, j):
        ir = idx_i.at[j]
        jr = idx_j.at[j]
        o = 7 * p
        for t in range(3):
            pltpu.async_copy(ubs[o + t], accs[t].at[ir], sss[p], add=True)
            pltpu.async_copy(ubs[o + 4 + t], accs[t].at[jr], sss[p], add=True)
        pltpu.async_copy(ubs[o + 3], accs[3].at[ir], sss[p], add=True)
        pltpu.async_copy(ubs[o + 3], accs[3].at[jr], sss[p], add=True)

    def wait_scatters(p):
        for t in range(8):
            pltpu.make_async_copy(dummy, ubs[7 * p], sss[p]).wait()

    def compute(p):
        o = 7 * p
        og = 8 * p
        for k in range(SB // L):
            sl = pl.ds(k * L, L)
            x0i = gb[og + 0][sl]; x1i = gb[og + 1][sl]
            u0i = gb[og + 2][sl]; u1i = gb[og + 3][sl]
            x0j = gb[og + 4][sl]; x1j = gb[og + 5][sl]
            u0j = gb[og + 6][sl]; u1j = gb[og + 7][sl]
            dx = x0j - x0i
            dy = x1j - x1i
            len2 = dx * dx + dy * dy + EPS
            r = _rsqrt(len2)
            wg = r * r
            du0 = u0j - u0i
            du1 = u1j - u1i
            divc = wg * r * (du0 * dx + du1 * dy)
            lap0 = wg * du0
            lap1 = wg * du1
            ubs[o + 0][sl] = divc
            ubs[o + 1][sl] = lap0
            ubs[o + 2][sl] = lap1
            ubs[o + 3][sl] = wg
            ubs[o + 4][sl] = -divc
            ubs[o + 5][sl] = -lap0
            ubs[o + 6][sl] = -lap1

    def _pipe_pair(jj, carry2):
        j = 2 * jj
        wait_gathers(0)
        issue_gathers(1, j + 1)

        @pl.when(jj > 0)
        def _():
            wait_scatters(0)

        compute(0)
        issue_scatters(0, j)

        wait_gathers(1)

        @pl.when(jj < CJ // 2 - 1)
        def _():
            issue_gathers(0, j + 2)

        @pl.when(jj > 0)
        def _():
            wait_scatters(1)

        compute(1)
        issue_scatters(1, j + 1)
        return carry2

    def edge_chunk(g, carry):
        h = g * NS + s

        @pl.when(h < NCH)
        def _chunk():
            pltpu.sync_copy(eip.at[c, pl.ds(h * CJ, CJ)], idx_i)
            pltpu.sync_copy(ejp.at[c, pl.ds(h * CJ, CJ)], idx_j)
            issue_gathers(0, 0)
            lax.fori_loop(0, CJ // 2, _pipe_pair, 0)
            wait_scatters(0)
            wait_scatters(1)

        return carry

    lax.fori_loop(0, NG, edge_chunk, 0)
    plsc.subcore_barrier()

    @pl.when(s < 15)
    def _rb_full():
        for t in range(4):
            pltpu.sync_copy(accs[t].at[pl.ds(s * FN, FN)], fins[t])

    @pl.when(s == 15)
    def _rb_last():
        for t in range(4):
            pltpu.sync_copy(accs[t].at[pl.ds(15 * FN, FLAST)],
                            fins[t].at[pl.ds(0, FLAST)])

    limit = jnp.where(s < 15, FN, FLAST)
    zero = jnp.zeros((L,), jnp.float32)

    def nodef(t, carry):
        sdv, slp = carry
        rows0 = t * L + iot
        valid = (rows0 < limit).astype(jnp.float32)
        sl = pl.ds(t * L, L)
        a0v = f0[sl]; a1v = f1[sl]; a2v = f2[sl]
        den = f3[sl] + EPS
        dv = a0v / den
        l0 = a1v / den
        l1 = a2v / den
        return (sdv + valid * dv * dv,
                slp + valid * (l0 * l0 + l1 * l1))

    sdv, slp = lax.fori_loop(0, FN // L, nodef, (zero, zero))

    base = jnp.minimum(wid, DWK - 1) * DW
    pltpu.sync_copy(xug.at[pl.ds(U0 + base, DW)], ub)
    pltpu.sync_copy(xug.at[pl.ds(G0 + base, DW)], gbv)

    def dataf(t, acc):
        dd = ub[pl.ds(t * L, L)] - gbv[pl.ds(t * L, L)]
        return acc + dd * dd

    sdat = lax.fori_loop(0, DW // L, dataf, zero)
    live = jnp.where(wid < DWK, 1.0, 0.0).astype(jnp.float32)

    pb[0] = sdv * (DIV_W / (B * N))
    pb[1] = slp * (LAP_W / (B * N * 2))
    pb[2] = sdat * live * (DATA_W / (B * N * 2))
    pltpu.sync_copy(pb, part_hbm.at[wid])


@functools.cache
def _build_sc_kernel():
    return pl.kernel(
        _sc_body_real,
        out_type=jax.ShapeDtypeStruct((NC * NS, 3, L), jnp.float32),
        mesh=plsc.VectorSubcoreMesh(
            core_axis_name="c", subcore_axis_name="s",
            num_cores=NC, num_subcores=NS),
        compiler_params=pltpu.CompilerParams(needs_layout_passes=False),
        scratch_types=(
            [pltpu.VMEM_SHARED((NP,), jnp.float32)] * 4
            + [pltpu.VMEM_SHARED((NP,), jnp.float32)] * 4
            + [pltpu.VMEM((CJ, SB), jnp.int32)] * 2
            + [pltpu.VMEM((SB,), jnp.float32)] * 16
            + [pltpu.VMEM((SB,), jnp.float32)] * 14
            + [pltpu.VMEM((2 * SG,), jnp.float32)]
            + [pltpu.VMEM((SG,), jnp.float32)] * 2
            + [pltpu.VMEM((FN,), jnp.float32)] * 4
            + [pltpu.VMEM((DW,), jnp.float32)] * 2
            + [pltpu.VMEM((3, L), jnp.float32)]
            + [pltpu.SemaphoreType.DMA] * 4
        ),
    )


def _sum_body(x_ref, o_ref):
    o_ref[0, 0] = jnp.sum(x_ref[...])


def _final_sum(x):
    return pl.pallas_call(
        _sum_body,
        out_shape=jax.ShapeDtypeStruct((1, 1), jnp.float32),
        out_specs=pl.BlockSpec(memory_space=pltpu.SMEM),
    )(x)


@jax.jit
def kernel(x, u, u_gt, edges):
    xug = jnp.concatenate(
        [x.reshape(-1), u.reshape(-1), u_gt.reshape(-1)])
    eip = jnp.pad(edges[..., 0].reshape(B, EIR, SB),
                  ((0, 0), (0, EIRP - EIR), (0, 0)), constant_values=N)
    ejp = jnp.pad(edges[..., 1].reshape(B, EIR, SB),
                  ((0, 0), (0, EIRP - EIR), (0, 0)), constant_values=N)
    zf = jnp.zeros((SG,), jnp.float32)
    part = _build_sc_kernel()(xug, eip, ejp, zf)
    return _final_sum(part.reshape(12, 128))[0, 0]


# speedup vs baseline: 5.4809x; 1.2668x over previous
import functools

import jax
import jax.numpy as jnp
from jax import lax
from jax.experimental import pallas as pl
from jax.experimental.pallas import tpu as pltpu
from jax.experimental.pallas import tpu_sc as plsc

B, N, E = 2, 50000, 800000
NC, NS, L = 2, 16, 16
SB = 128
CJ = 16
EIR = E // SB
NP = N + 8
NCH = EIR // CJ
REMG = EIR - NCH * CJ
NG = -(-NCH // NS)
SG = N // 10
FN = 3200
FLAST = N - 15 * FN
DW = 8000
DWK = (B * N * 2) // DW
EPS = 1e-8
DIV_W, LAP_W, DATA_W = 1.0, 0.1, 1.0


X0, U0, G0 = 0, 2 * B * N, 4 * B * N


def _rsqrt(z):
    ii = lax.bitcast_convert_type(z, jnp.int32)
    ii = jnp.int32(0x5F3759DF) - (ii >> 1)
    y = lax.bitcast_convert_type(ii, jnp.float32)
    for _ in range(2):
        y = y * (1.5 - 0.5 * z * y * y)
    return y


def _sc_body_real(xug, epj, zf, part_hbm,
                  t0, t1, t2, t3, a0, a1, a2, a3,
                  ebv,
                  g0, g1, g2, g3, g4, g5, g6, g7,
                  g8, g9, g10, g11, g12, g13, g14, g15,
                  b0, b1, b2, b3, b4, b5, b6,
                  b7, b8, b9, b10, b11, b12, b13,
                  stg, d0, d1, f0, f1, f2, f3, ub, gbv, pb,
                  sg0, sg1, ss0, ss1):
    c = lax.axis_index("c")
    s = lax.axis_index("s")
    wid = c * NS + s
    iot = lax.iota(jnp.int32, L)
    tabs = (t0, t1, t2, t3)
    accs = (a0, a1, a2, a3)
    gb = (g0, g1, g2, g3, g4, g5, g6, g7,
          g8, g9, g10, g11, g12, g13, g14, g15)
    ubs = (b0, b1, b2, b3, b4, b5, b6,
           b7, b8, b9, b10, b11, b12, b13)
    fins = (f0, f1, f2, f3)

    def _deint(n):
        def body(k, carry):
            pos = 2 * (k * L) + 2 * iot
            d0[pl.ds(k * L, L)] = plsc.load_gather(stg, [pos])
            d1[pl.ds(k * L, L)] = plsc.load_gather(stg, [pos + 1])
            return carry

        lax.fori_loop(0, n // L, body, 0)
        tail = n - (n // L) * L
        if tail:
            rows0 = (n // L) * L + iot
            valid = rows0 < n
            rows = jnp.minimum(rows0, n - 1)
            v0 = plsc.load_gather(stg, [2 * rows])
            v1 = plsc.load_gather(stg, [2 * rows + 1])
            plsc.store_scatter(d0, [rows], v0, mask=valid)
            plsc.store_scatter(d1, [rows], v1, mask=valid)

    @pl.when(s < 10)
    def _stage():
        pltpu.sync_copy(xug.at[pl.ds(X0 + 2 * (c * N + s * SG), 2 * SG)], stg)
        _deint(SG)
        pltpu.sync_copy(d0, t0.at[pl.ds(s * SG, SG)])
        pltpu.sync_copy(d1, t1.at[pl.ds(s * SG, SG)])
        pltpu.sync_copy(xug.at[pl.ds(U0 + 2 * (c * N + s * SG), 2 * SG)], stg)
        _deint(SG)
        pltpu.sync_copy(d0, t2.at[pl.ds(s * SG, SG)])
        pltpu.sync_copy(d1, t3.at[pl.ds(s * SG, SG)])
        pltpu.sync_copy(zf, d0)
        for t in range(4):
            pltpu.sync_copy(d0, accs[t].at[pl.ds(s * SG, SG)])

    @pl.when(s == 10)
    def _stage_pad():
        pltpu.sync_copy(zf, d0)
        for t in range(4):
            pltpu.sync_copy(d0.at[pl.ds(0, 8)], tabs[t].at[pl.ds(N, 8)])
        for t in range(4):
            pltpu.sync_copy(d0.at[pl.ds(0, 8)], accs[t].at[pl.ds(N, 8)])

    plsc.subcore_barrier()

    sgs = (sg0, sg1)
    sss = (ss0, ss1)
    dummy = xug.at[pl.ds(0, SB)]

    def issue_gathers(p, j):
        ir = ebv.at[2 * j]
        jr = ebv.at[2 * j + 1]
        for t in range(4):
            pltpu.async_copy(tabs[t].at[ir], gb[8 * p + t], sgs[p])
            pltpu.async_copy(tabs[t].at[jr], gb[8 * p + 4 + t], sgs[p])

    def wait_gathers(p):
        for t in range(8):
            pltpu.make_async_copy(dummy, gb[8 * p + t], sgs[p]).wait()

    def issue_scatters(p, j):
        ir = ebv.at[2 * j]
        jr = ebv.at[2 * j + 1]
        o = 7 * p
        for t in range(3):
            pltpu.async_copy(ubs[o + t], accs[t].at[ir], sss[p], add=True)
            pltpu.async_copy(ubs[o + 4 + t], accs[t].at[jr], sss[p], add=True)
        pltpu.async_copy(ubs[o + 3], accs[3].at[ir], sss[p], add=True)
        pltpu.async_copy(ubs[o + 3], accs[3].at[jr], sss[p], add=True)

    def wait_scatters(p):
        for t in range(8):
            pltpu.make_async_copy(dummy, ubs[7 * p], sss[p]).wait()

    def compute(p):
        o = 7 * p
        og = 8 * p
        for k in range(SB // L):
            sl = pl.ds(k * L, L)
            x0i = gb[og + 0][sl]; x1i = gb[og + 1][sl]
            u0i = gb[og + 2][sl]; u1i = gb[og + 3][sl]
            x0j = gb[og + 4][sl]; x1j = gb[og + 5][sl]
            u0j = gb[og + 6][sl]; u1j = gb[og + 7][sl]
            dx = x0j - x0i
            dy = x1j - x1i
            len2 = dx * dx + dy * dy + EPS
            r = _rsqrt(len2)
            wg = r * r
            du0 = u0j - u0i
            du1 = u1j - u1i
            divc = wg * r * (du0 * dx + du1 * dy)
            lap0 = wg * du0
            lap1 = wg * du1
            ubs[o + 0][sl] = divc
            ubs[o + 1][sl] = lap0
            ubs[o + 2][sl] = lap1
            ubs[o + 3][sl] = wg
            ubs[o + 4][sl] = -divc
            ubs[o + 5][sl] = -lap0
            ubs[o + 6][sl] = -lap1

    def _pipe_pair(jj, carry2):
        j = 2 * jj
        wait_gathers(0)
        issue_gathers(1, j + 1)

        @pl.when(jj > 0)
        def _():
            wait_scatters(0)

        compute(0)
        issue_scatters(0, j)

        wait_gathers(1)

        @pl.when(jj < CJ // 2 - 1)
        def _():
            issue_gathers(0, j + 2)

        @pl.when(jj > 0)
        def _():
            wait_scatters(1)

        compute(1)
        issue_scatters(1, j + 1)
        return carry2

    def edge_chunk(g, carry):
        h = g * NS + s

        @pl.when(h < NCH)
        def _chunk():
            pltpu.sync_copy(epj.at[c, pl.ds(h * 2 * CJ, 2 * CJ)], ebv)
            issue_gathers(0, 0)
            lax.fori_loop(0, CJ // 2, _pipe_pair, 0)
            wait_scatters(0)
            wait_scatters(1)

        return carry

    lax.fori_loop(0, NG, edge_chunk, 0)

    @pl.when(s == 15)
    def _rem():
        r0 = NCH * 2 * CJ
        pltpu.sync_copy(epj.at[c, pl.ds(r0, 16)], ebv.at[pl.ds(0, 16)])
        pltpu.sync_copy(epj.at[c, pl.ds(r0 + 16, 2 * REMG - 16)],
                        ebv.at[pl.ds(16, 2 * REMG - 16)])
        for j2 in range(REMG):
            issue_gathers(0, j2)
            wait_gathers(0)
            compute(0)
            issue_scatters(0, j2)
            wait_scatters(0)

    plsc.subcore_barrier()

    @pl.when(s < 15)
    def _rb_full():
        for t in range(4):
            pltpu.sync_copy(accs[t].at[pl.ds(s * FN, FN)], fins[t])

    @pl.when(s == 15)
    def _rb_last():
        for t in range(4):
            pltpu.sync_copy(accs[t].at[pl.ds(15 * FN, FLAST)],
                            fins[t].at[pl.ds(0, FLAST)])

    limit = jnp.where(s < 15, FN, FLAST)
    zero = jnp.zeros((L,), jnp.float32)

    def nodef(t, carry):
        sdv, slp = carry
        rows0 = t * L + iot
        valid = (rows0 < limit).astype(jnp.float32)
        sl = pl.ds(t * L, L)
        a0v = f0[sl]; a1v = f1[sl]; a2v = f2[sl]
        den = f3[sl] + EPS
        dv = a0v / den
        l0 = a1v / den
        l1 = a2v / den
        return (sdv + valid * dv * dv,
                slp + valid * (l0 * l0 + l1 * l1))

    sdv, slp = lax.fori_loop(0, FN // L, nodef, (zero, zero))

    base = jnp.minimum(wid, DWK - 1) * DW
    pltpu.sync_copy(xug.at[pl.ds(U0 + base, DW)], ub)
    pltpu.sync_copy(xug.at[pl.ds(G0 + base, DW)], gbv)

    def dataf(t, acc):
        dd = ub[pl.ds(t * L, L)] - gbv[pl.ds(t * L, L)]
        return acc + dd * dd

    sdat = lax.fori_loop(0, DW // L, dataf, zero)
    live = jnp.where(wid < DWK, 1.0, 0.0).astype(jnp.float32)

    pb[0] = sdv * (DIV_W / (B * N))
    pb[1] = slp * (LAP_W / (B * N * 2))
    pb[2] = sdat * live * (DATA_W / (B * N * 2))
    pltpu.sync_copy(pb, part_hbm.at[wid])


@functools.cache
def _build_sc_kernel():
    return pl.kernel(
        _sc_body_real,
        out_type=jax.ShapeDtypeStruct((NC * NS, 3, L), jnp.float32),
        mesh=plsc.VectorSubcoreMesh(
            core_axis_name="c", subcore_axis_name="s",
            num_cores=NC, num_subcores=NS),
        compiler_params=pltpu.CompilerParams(needs_layout_passes=False),
        scratch_types=(
            [pltpu.VMEM_SHARED((NP,), jnp.float32)] * 4
            + [pltpu.VMEM_SHARED((NP,), jnp.float32)] * 4
            + [pltpu.VMEM((2 * CJ, 128), jnp.int32)]
            + [pltpu.VMEM((SB,), jnp.float32)] * 16
            + [pltpu.VMEM((SB,), jnp.float32)] * 14
            + [pltpu.VMEM((2 * SG,), jnp.float32)]
            + [pltpu.VMEM((SG,), jnp.float32)] * 2
            + [pltpu.VMEM((FN,), jnp.float32)] * 4
            + [pltpu.VMEM((DW,), jnp.float32)] * 2
            + [pltpu.VMEM((3, L), jnp.float32)]
            + [pltpu.SemaphoreType.DMA] * 4
        ),
    )


def _sum_body(x_ref, o_ref):
    o_ref[0, 0] = jnp.sum(x_ref[...])


def _final_sum(x):
    return pl.pallas_call(
        _sum_body,
        out_shape=jax.ShapeDtypeStruct((1, 1), jnp.float32),
        out_specs=pl.BlockSpec(memory_space=pltpu.SMEM),
    )(x)


@jax.jit
def kernel(x, u, u_gt, edges):
    xug = jnp.concatenate(
        [x.reshape(-1), u.reshape(-1), u_gt.reshape(-1)])
    epj = (edges.transpose(0, 2, 1)
           .reshape(B, 2, EIR, SB)
           .transpose(0, 2, 1, 3)
           .reshape(B, 2 * EIR, SB))
    zf = jnp.zeros((SG,), jnp.float32)
    part = _build_sc_kernel()(xug, epj, zf)
    return _final_sum(part.reshape(12, 128))[0, 0]
